# baseline (speedup 1.0000x reference)
import jax
import jax.numpy as jnp
from jax import lax
from jax.experimental import pallas as pl
from jax.experimental.pallas import tpu as pltpu
from jax.experimental.pallas import tpu_sc as plsc

N_WORDS = 1000000
D = 64
B = 16384
L = 16
NC = 2
NS = 16
NW = NC * NS
BPW = B // NW
CH = 128
NCH = BPW // CH
NV = D // L
HB = BPW // 2


def _body(wids_hbm, cids_hbm, woff_hbm, coff_hbm, table_hbm, params_hbm,
          out_hbm, idx_w, idx_c, offw, offc, wrows, crows, pvec, outv, tr,
          sem):
  wid = lax.axis_index("s") * NC + lax.axis_index("c")
  base = wid * BPW

  pltpu.sync_copy(wids_hbm.at[wid], idx_w)
  pltpu.sync_copy(cids_hbm.at[wid], idx_c)
  pltpu.sync_copy(woff_hbm.at[wid], offw)
  pltpu.sync_copy(coff_hbm.at[wid], offc)
  pltpu.sync_copy(params_hbm, pvec)

  wbv = pvec[0, :]
  cbv = pvec[1, :]
  fcbv = pvec[2, :]
  fw = [pvec[4 + i, :] for i in range(NV)]
  lane = lax.broadcasted_iota(jnp.int32, (L,), 0)

  for h in range(2):
    copies = []
    for j in range(HB // CH):
      jj = h * (HB // CH) + j
      copies.append(pltpu.async_copy(
          table_hbm.at[idx_w.at[jj]], wrows.at[pl.ds(j * CH, CH)], sem))
      copies.append(pltpu.async_copy(
          table_hbm.at[idx_c.at[jj]], crows.at[pl.ds(j * CH, CH)], sem))
    for c in copies:
      c.wait()

    def group(g, _):
      ovw = offw[pl.ds(h * HB + g * L, L)]
      ovc = offc[pl.ds(h * HB + g * L, L)]
      for k in range(L):
        e = g * L + k
        ow = ovw[k]
        oc = ovc[k]
        acc = None
        for i in range(NV):
          w = wrows[e, pl.ds(ow + i * L, L)]
          c = crows[e, pl.ds(oc + i * L, L)]
          t = ((w + wbv) * (c + cbv)) * fw[i]
          acc = t if acc is None else acc + t
        plsc.store_scatter(tr, [lane, jnp.full((L,), k, jnp.int32)], acc)
      zv = None
      for j in range(L):
        r = tr[j, pl.ds(0, L)]
        zv = r if zv is None else zv + r
      x = zv + fcbv
      outv[pl.ds(h * HB + g * L, L)] = 1.0 / (1.0 + jnp.exp(-x))
      return _

    lax.fori_loop(0, HB // L, group, None)

  pltpu.sync_copy(outv, out_hbm.at[pl.ds(base, BPW)])


@jax.jit
def _run(wids3, cids3, woff, coff, table2, params):
  mesh = plsc.VectorSubcoreMesh(
      core_axis_name="c", subcore_axis_name="s",
      num_cores=NC, num_subcores=NS)
  return pl.kernel(
      _body,
      out_type=jax.ShapeDtypeStruct((B,), jnp.float32),
      mesh=mesh,
      compiler_params=pltpu.CompilerParams(
          needs_layout_passes=False, use_tc_tiling_on_sc=True),
      scratch_types=[
          pltpu.VMEM((NCH, CH), jnp.int32),
          pltpu.VMEM((NCH, CH), jnp.int32),
          pltpu.VMEM((BPW,), jnp.int32),
          pltpu.VMEM((BPW,), jnp.int32),
          pltpu.VMEM((HB, 2 * D), jnp.float32),
          pltpu.VMEM((HB, 2 * D), jnp.float32),
          pltpu.VMEM((8, L), jnp.float32),
          pltpu.VMEM((BPW,), jnp.float32),
          pltpu.VMEM((L, L + 1), jnp.float32),
          pltpu.SemaphoreType.DMA,
      ],
  )(wids3, cids3, woff, coff, table2, params)


def kernel(word_ids, context_ids, table, fc_w, fc_b, word_bias, con_bias):
  wi = word_ids.astype(jnp.int32)
  ci = context_ids.astype(jnp.int32)
  wids3 = (wi // 2).reshape(NW, NCH, CH)
  cids3 = (ci // 2).reshape(NW, NCH, CH)
  woff = ((wi % 2) * D).reshape(NW, BPW)
  coff = ((ci % 2) * D).reshape(NW, BPW)
  table2 = table.reshape(N_WORDS // 2, 2 * D)
  params = jnp.concatenate([
      jnp.broadcast_to(word_bias.astype(jnp.float32), (L,)),
      jnp.broadcast_to(con_bias.astype(jnp.float32), (L,)),
      jnp.broadcast_to(fc_b.astype(jnp.float32), (L,)),
      jnp.zeros((L,), jnp.float32),
      fc_w.astype(jnp.float32).reshape(D),
  ]).reshape(8, L)
  out = _run(wids3, cids3, woff, coff, table2, params)
  return out.reshape(B, 1)


# speedup vs baseline: 1.6720x; 1.6720x over previous
import jax
import jax.numpy as jnp
from jax import lax
from jax.experimental import pallas as pl
from jax.experimental.pallas import tpu as pltpu
from jax.experimental.pallas import tpu_sc as plsc

N_WORDS = 1000000
D = 64
B = 16384
L = 16
NC = 2
NS = 16
NW = NC * NS
BPW = B // NW
NG = BPW // L
NV = D // L


def _body(wids_hbm, cids_hbm, table_hbm, params_hbm, out_hbm,
          idx_w, idx_c, wrows, crows, pvec, outv, tr, sem0, sem1):
  wid = lax.axis_index("s") * NC + lax.axis_index("c")
  base = wid * BPW

  pltpu.sync_copy(wids_hbm.at[wid], idx_w)
  pltpu.sync_copy(cids_hbm.at[wid], idx_c)
  pltpu.sync_copy(params_hbm, pvec)

  wbv = pvec[0, :]
  cbv = pvec[1, :]
  fcbv = pvec[2, :]
  fw = [pvec[4 + i, :] for i in range(NV)]
  lane = lax.broadcasted_iota(jnp.int32, (L,), 0)
  sems = [sem0, sem1]

  def prefetch(g, buf):
    ivw = idx_w[pl.ds(g * L, L)]
    ivc = idx_c[pl.ds(g * L, L)]
    for k in range(L):
      pltpu.async_copy(table_hbm.at[ivw[k]], wrows.at[buf, k], sems[buf])
      pltpu.async_copy(table_hbm.at[ivc[k]], crows.at[buf, k], sems[buf])

  def drain(buf):
    pltpu.make_async_copy(
        table_hbm.at[pl.ds(0, L)], wrows.at[buf], sems[buf]).wait()
    pltpu.make_async_copy(
        table_hbm.at[pl.ds(0, L)], crows.at[buf], sems[buf]).wait()

  def compute(g, buf):
    for k in range(L):
      acc = None
      for i in range(NV):
        w = wrows[buf, k, pl.ds(i * L, L)]
        c = crows[buf, k, pl.ds(i * L, L)]
        t = ((w + wbv) * (c + cbv)) * fw[i]
        acc = t if acc is None else acc + t
      plsc.store_scatter(tr, [lane, jnp.full((L,), k, jnp.int32)], acc)
    zv = None
    for j in range(L):
      r = tr[j, pl.ds(0, L)]
      zv = r if zv is None else zv + r
    x = zv + fcbv
    outv[pl.ds(g * L, L)] = 1.0 / (1.0 + jnp.exp(-x))

  prefetch(0, 0)

  def step(h, _):
    g0 = h * 2
    g1 = g0 + 1
    prefetch(g1, 1)
    drain(0)
    compute(g0, 0)

    @pl.when(g1 + 1 < NG)
    def _():
      prefetch(g1 + 1, 0)

    drain(1)
    compute(g1, 1)
    return _

  lax.fori_loop(0, NG // 2, step, None)

  pltpu.sync_copy(outv, out_hbm.at[pl.ds(base, BPW)])


@jax.jit
def _run(wids2, cids2, table, params):
  mesh = plsc.VectorSubcoreMesh(
      core_axis_name="c", subcore_axis_name="s",
      num_cores=NC, num_subcores=NS)
  return pl.kernel(
      _body,
      out_type=jax.ShapeDtypeStruct((B,), jnp.float32),
      mesh=mesh,
      compiler_params=pltpu.CompilerParams(
          needs_layout_passes=False, use_tc_tiling_on_sc=True),
      scratch_types=[
          pltpu.VMEM((BPW,), jnp.int32),
          pltpu.VMEM((BPW,), jnp.int32),
          pltpu.VMEM((2, L, D), jnp.float32),
          pltpu.VMEM((2, L, D), jnp.float32),
          pltpu.VMEM((8, L), jnp.float32),
          pltpu.VMEM((BPW,), jnp.float32),
          pltpu.VMEM((L, L + 1), jnp.float32),
          pltpu.SemaphoreType.DMA,
          pltpu.SemaphoreType.DMA,
      ],
  )(wids2, cids2, table, params)


def kernel(word_ids, context_ids, table, fc_w, fc_b, word_bias, con_bias):
  wids2 = word_ids.astype(jnp.int32).reshape(NW, BPW)
  cids2 = context_ids.astype(jnp.int32).reshape(NW, BPW)
  params = jnp.concatenate([
      jnp.broadcast_to(word_bias.astype(jnp.float32), (L,)),
      jnp.broadcast_to(con_bias.astype(jnp.float32), (L,)),
      jnp.broadcast_to(fc_b.astype(jnp.float32), (L,)),
      jnp.zeros((L,), jnp.float32),
      fc_w.astype(jnp.float32).reshape(D),
  ]).reshape(8, L)
  out = _run(wids2, cids2, table, params)
  return out.reshape(B, 1)
